# baseline (speedup 1.0000x reference)
import functools

import jax
import jax.numpy as jnp
from jax import lax
from jax.experimental import pallas as pl
from jax.experimental.pallas import tpu as pltpu
from jax.experimental.pallas import tpu_sc as plsc

LANES = 16
IDX_CHUNK = 128


@functools.cache
def _make_sc_kernel(batch: int, n_factors: int):
    info = plsc.get_sparse_core_info()
    nc, ns = info.num_cores, info.num_subcores
    nw = nc * ns
    assert batch % (nw * LANES) == 0
    b_per_w = batch // nw
    n_chunks = b_per_w // IDX_CHUNK
    mesh = plsc.VectorSubcoreMesh(core_axis_name="c", subcore_axis_name="s")

    @functools.partial(
        pl.kernel,
        mesh=mesh,
        compiler_params=pltpu.CompilerParams(
            needs_layout_passes=False, use_tc_tiling_on_sc=False),
        out_type=jax.ShapeDtypeStruct((batch,), jnp.float32),
        scratch_types=[
            pltpu.VMEM((n_chunks, IDX_CHUNK), jnp.int32),
            pltpu.VMEM((n_chunks, IDX_CHUNK), jnp.int32),
            pltpu.VMEM((b_per_w, n_factors), jnp.float32),
            pltpu.VMEM((b_per_w, n_factors), jnp.float32),
            pltpu.VMEM((b_per_w,), jnp.float32),
            pltpu.SemaphoreType.DMA,
        ],
    )
    def sc_kernel(users_h, items_h, theta_h, beta_h, out_h,
                  uidx, iidx, trows, brows, res, sem):
        wid = lax.axis_index("s") * nc + lax.axis_index("c")
        base = wid * b_per_w

        copies = []
        for j in range(n_chunks):
            off = base + j * IDX_CHUNK
            copies.append(pltpu.async_copy(
                users_h.at[pl.ds(off, IDX_CHUNK)], uidx.at[j], sem))
            copies.append(pltpu.async_copy(
                items_h.at[pl.ds(off, IDX_CHUNK)], iidx.at[j], sem))
        for c in copies:
            c.wait()

        gathers = []
        for j in range(n_chunks):
            dst = pl.ds(j * IDX_CHUNK, IDX_CHUNK)
            gathers.append(pltpu.async_copy(
                theta_h.at[uidx.at[j]], trows.at[dst], sem))
            gathers.append(pltpu.async_copy(
                beta_h.at[iidx.at[j]], brows.at[dst], sem))
        for c in gathers:
            c.wait()

        n_half = n_factors // LANES
        iota = lax.iota(jnp.int32, LANES)

        def group(g, carry):
            r = jnp.zeros((LANES,), jnp.float32)
            for k in range(LANES):
                i = g * LANES + k
                p = jnp.zeros((LANES,), jnp.float32)
                for h in range(n_half):
                    a = trows[i, pl.ds(h * LANES, LANES)]
                    b = brows[i, pl.ds(h * LANES, LANES)]
                    p = p + a * b
                r = jnp.where(iota == k, jnp.sum(p), r)
            res[pl.ds(g * LANES, LANES)] = 1.0 / (1.0 + jnp.exp(-r))
            return carry

        lax.fori_loop(0, b_per_w // LANES, group, 0)
        pltpu.sync_copy(res, out_h.at[pl.ds(base, b_per_w)])

    return sc_kernel


def kernel(users, items, contexts, theta, beta):
    del contexts
    sc = _make_sc_kernel(users.shape[0], theta.shape[1])
    return sc(users.astype(jnp.int32), items.astype(jnp.int32), theta, beta)
